# Initial kernel scaffold; baseline (speedup 1.0000x reference)
import jax
import jax.numpy as jnp
from jax.experimental import pallas as pl


def kernel(x, tok_table, pos_table):
    raise NotImplementedError("write your pallas kernel here")


# speedup vs baseline: 2.0137x; 2.0137x over previous
import functools

import jax
import jax.numpy as jnp
from jax import lax
from jax.experimental import pallas as pl
from jax.experimental.pallas import tpu as pltpu
from jax.experimental.pallas import tpu_sc as plsc

_LANES = 16
_CH = 128


def _positions_body(x_ref, mask_ref, pos_ref):
    xb = x_ref[...]
    L = xb.shape[-1]
    mask = xb == 0
    nz = jnp.where(mask, 0.0, 1.0).astype(jnp.float32)
    ii = lax.broadcasted_iota(jnp.int32, (L, L), 0)
    jj = lax.broadcasted_iota(jnp.int32, (L, L), 1)
    tri = (ii <= jj).astype(jnp.float32)
    pos = jnp.dot(nz, tri, preferred_element_type=jnp.float32)
    mask_ref[...] = mask
    pos_ref[...] = jnp.where(mask, 0, pos.astype(jnp.int32))


def _make_positions(B, L, block_rows):
    grid = (B // block_rows,)
    return pl.pallas_call(
        _positions_body,
        grid=grid,
        in_specs=[pl.BlockSpec((block_rows, L), lambda i: (i, 0))],
        out_specs=[
            pl.BlockSpec((block_rows, L), lambda i: (i, 0)),
            pl.BlockSpec((block_rows, L), lambda i: (i, 0)),
        ],
        out_shape=[
            jax.ShapeDtypeStruct((B, L), jnp.bool_),
            jax.ShapeDtypeStruct((B, L), jnp.int32),
        ],
    )


def _make_sc_combine(N, D):
    info = plsc.get_sparse_core_info()
    NC, NS = info.num_cores, info.num_subcores
    NW = NC * NS
    assert N % (NW * _CH) == 0
    n_chunks = N // (NW * _CH)
    mesh = plsc.VectorSubcoreMesh(core_axis_name="c", subcore_axis_name="s")

    @functools.partial(
        pl.kernel,
        mesh=mesh,
        compiler_params=pltpu.CompilerParams(use_tc_tiling_on_sc=False),
        out_type=jax.ShapeDtypeStruct((N, D), jnp.float32),
        scratch_types=[
            pltpu.VMEM((n_chunks, _CH), jnp.int32),
            pltpu.VMEM((n_chunks, _CH), jnp.int32),
            pltpu.VMEM((_CH, D), jnp.float32),
            pltpu.VMEM((_CH, D), jnp.float32),
            pltpu.SemaphoreType.DMA,
            pltpu.SemaphoreType.DMA,
        ],
    )
    def sc_combine(xi_hbm, pi_hbm, tok_hbm, pos_hbm, out_hbm,
                   xi_v, pi_v, tr_v, pr_v, sem_t, sem_p):
        wid = lax.axis_index("s") * NC + lax.axis_index("c")
        pltpu.sync_copy(xi_hbm.at[pl.ds(wid * n_chunks, n_chunks)], xi_v)
        pltpu.sync_copy(pi_hbm.at[pl.ds(wid * n_chunks, n_chunks)], pi_v)

        def chunk(j, carry):
            ct = pltpu.async_copy(tok_hbm.at[xi_v.at[j]], tr_v, sem_t)
            cp = pltpu.async_copy(pos_hbm.at[pi_v.at[j]], pr_v, sem_p)
            ct.wait()
            cp.wait()

            def row(r, c2):
                for cc in range(D // _LANES):
                    sl = pl.ds(cc * _LANES, _LANES)
                    tr_v[r, sl] = tr_v[r, sl] + pr_v[r, sl]
                return c2

            lax.fori_loop(0, _CH, row, 0)
            pltpu.sync_copy(
                tr_v, out_hbm.at[pl.ds((wid * n_chunks + j) * _CH, _CH)])
            return carry

        lax.fori_loop(0, n_chunks, chunk, 0)

    return sc_combine


def kernel(x, tok_table, pos_table):
    B, L = x.shape
    V, D = tok_table.shape
    x32 = x.astype(jnp.int32)

    mask, positions = _make_positions(B, L, 512)(x32)

    N = B * L
    xi = x32.reshape(N // _CH, _CH)
    pi = positions.reshape(N // _CH, _CH)
    out = _make_sc_combine(N, D)(xi, pi, tok_table, pos_table)
    return out.reshape(B, L, D), mask


# speedup vs baseline: 2.0170x; 1.0017x over previous
import functools

import jax
import jax.numpy as jnp
from jax import lax
from jax.experimental import pallas as pl
from jax.experimental.pallas import tpu as pltpu
from jax.experimental.pallas import tpu_sc as plsc

_LANES = 16
_CH = 128


def _positions_body(x_ref, mask_ref, pos_ref):
    xb = x_ref[...]
    L = xb.shape[-1]
    mask = xb == 0
    nz = jnp.where(mask, 0.0, 1.0).astype(jnp.float32)
    ii = lax.broadcasted_iota(jnp.int32, (L, L), 0)
    jj = lax.broadcasted_iota(jnp.int32, (L, L), 1)
    tri = (ii <= jj).astype(jnp.float32)
    pos = jnp.dot(nz, tri, preferred_element_type=jnp.float32)
    mask_ref[...] = mask
    pos_ref[...] = jnp.where(mask, 0, pos.astype(jnp.int32))


def _make_positions(B, L, block_rows):
    grid = (B // block_rows,)
    return pl.pallas_call(
        _positions_body,
        grid=grid,
        in_specs=[pl.BlockSpec((block_rows, L), lambda i: (i, 0))],
        out_specs=[
            pl.BlockSpec((block_rows, L), lambda i: (i, 0)),
            pl.BlockSpec((block_rows, L), lambda i: (i, 0)),
        ],
        out_shape=[
            jax.ShapeDtypeStruct((B, L), jnp.bool_),
            jax.ShapeDtypeStruct((B, L), jnp.int32),
        ],
    )


_NBUF = 4
_NOB = 2
_GROUP = 40


def _make_sc_combine(N, D, VP):
    info = plsc.get_sparse_core_info()
    NC, NS = info.num_cores, info.num_subcores
    NW = NC * NS
    assert N % (NW * _CH) == 0
    n_chunks = N // (NW * _CH)
    assert n_chunks % _GROUP == 0 and _GROUP % _NBUF == 0
    NG = n_chunks // _GROUP
    NSUP = _GROUP // _NBUF
    mesh = plsc.VectorSubcoreMesh(core_axis_name="c", subcore_axis_name="s")

    @functools.partial(
        pl.kernel,
        mesh=mesh,
        compiler_params=pltpu.CompilerParams(use_tc_tiling_on_sc=False),
        out_type=jax.ShapeDtypeStruct((N, D), jnp.float32),
        scratch_types=[
            pltpu.VMEM((_GROUP, _CH), jnp.int32),
            pltpu.VMEM((_GROUP, _CH), jnp.int32),
            pltpu.VMEM((_NBUF, _CH, D), jnp.float32),
            pltpu.VMEM((_NBUF, _CH, D), jnp.float32),
            pltpu.VMEM((_NOB, _CH, D), jnp.float32),
            [pltpu.SemaphoreType.DMA] * _NBUF,
            [pltpu.SemaphoreType.DMA] * _NOB,
        ],
    )
    def sc_combine(xi_hbm, pi_hbm, tok_hbm, pos_hbm, out_hbm,
                   xi_v, pi_v, tr, pr, ob, sg, so):
        sid = lax.axis_index("s")
        wid = sid * NC + lax.axis_index("c")
        base_chunk = wid * n_chunks

        def group_body(g, carry):
            goff = base_chunk + g * _GROUP
            pltpu.sync_copy(xi_hbm.at[pl.ds(goff, _GROUP)], xi_v)
            pltpu.sync_copy(pi_hbm.at[pl.ds(goff, _GROUP)], pi_v)
            for b in range(_NBUF):
                pltpu.async_copy(tok_hbm.at[xi_v.at[b]], tr.at[b], sg[b])
                pltpu.async_copy(pos_hbm.at[pi_v.at[b]], pr.at[b], sg[b])

            def super_body(jj, carry2):
                for b in range(_NBUF):
                    j = jj * _NBUF + b
                    jg = g * _GROUP + j
                    b2 = b % _NOB
                    pltpu.make_async_copy(
                        tok_hbm.at[xi_v.at[b]], tr.at[b], sg[b]).wait()
                    pltpu.make_async_copy(
                        pos_hbm.at[pi_v.at[b]], pr.at[b], sg[b]).wait()

                    @pl.when(jg >= _NOB)
                    def _():
                        pltpu.make_async_copy(
                            ob.at[b2], out_hbm.at[pl.ds(0, _CH)],
                            so[b2]).wait()

                    def row(r, cr):
                        for cc in range(D // _LANES):
                            sl = pl.ds(cc * _LANES, _LANES)
                            ob[b2, r, sl] = tr[b, r, sl] + pr[b, r, sl]
                        return cr

                    lax.fori_loop(0, _CH, row, 0, unroll=4)
                    pltpu.async_copy(
                        ob.at[b2], out_hbm.at[pl.ds((goff + j) * _CH, _CH)],
                        so[b2])

                    @pl.when(j + _NBUF < _GROUP)
                    def _():
                        jn = j + _NBUF
                        pltpu.async_copy(
                            tok_hbm.at[xi_v.at[jn]], tr.at[b], sg[b])
                        pltpu.async_copy(
                            pos_hbm.at[pi_v.at[jn]], pr.at[b], sg[b])
                return carry2

            lax.fori_loop(0, NSUP, super_body, 0)
            return carry

        lax.fori_loop(0, NG, group_body, 0)
        for b2 in range(_NOB):
            pltpu.make_async_copy(
                ob.at[b2], out_hbm.at[pl.ds(0, _CH)], so[b2]).wait()

    return sc_combine


def kernel(x, tok_table, pos_table):
    B, L = x.shape
    V, D = tok_table.shape
    x32 = x.astype(jnp.int32)

    mask, positions = _make_positions(B, L, 512)(x32)

    N = B * L
    xi = x32.reshape(N // _CH, _CH)
    pi = positions.reshape(N // _CH, _CH)
    out = _make_sc_combine(N, D, pos_table.shape[0])(
        xi, pi, tok_table, pos_table)
    return out.reshape(B, L, D), mask
